# baseline (speedup 1.0000x reference)
import functools

import jax
import jax.numpy as jnp
from jax import lax
from jax.experimental import pallas as pl
from jax.experimental.pallas import tpu as pltpu
from jax.experimental.pallas import tpu_sc as plsc

B = 4096
L = 50
EMB = 64
NUM_SENS = 2
VOCAB = 1000000

NC = 2
NS = 16
NW = NC * NS
IPW = B * L // NW
TILE = 128
TPW = IPW // TILE

PROJ_R = 40960
PROJ_STEPS = -(-VOCAB // PROJ_R)

_sc_mesh = plsc.VectorSubcoreMesh(core_axis_name="c", subcore_axis_name="s")


def _proj_body(wt_ref, lin_w_ref, idxt_ref, d_ref, idx_out_ref):
    dw = (lin_w_ref[1, :] - lin_w_ref[0, :]).reshape(1, EMB)
    res = lax.dot_general(
        dw, wt_ref[...],
        (((1,), (0,)), ((), ())),
        preferred_element_type=jnp.float32,
    )
    d_ref[...] = res.reshape(d_ref.shape)

    @pl.when(pl.program_id(0) == 0)
    def _():
        for k in range(B // TILE):
            idx_out_ref[:, k, :] = idxt_ref[:, k * TILE:(k + 1) * TILE]


_proj = pl.pallas_call(
    _proj_body,
    grid=(PROJ_STEPS,),
    in_specs=[
        pl.BlockSpec((EMB, PROJ_R), lambda i: (0, i)),
        pl.BlockSpec((NUM_SENS, EMB), lambda i: (0, 0)),
        pl.BlockSpec((L, B), lambda i: (0, 0)),
    ],
    out_specs=(
        pl.BlockSpec((PROJ_R,), lambda i: (i,)),
        pl.BlockSpec((L, B // TILE, TILE), lambda i: (0, 0, 0)),
    ),
    out_shape=(
        jax.ShapeDtypeStruct((VOCAB,), jnp.float32),
        jax.ShapeDtypeStruct((L, B // TILE, TILE), jnp.int32),
    ),
)


RPW = B // NW
NGRP = RPW // 16


def _make_gather_body(wid_fn):
    def _gather_body(idx_hbm, d_hbm, out_hbm, idx_v, g_v, acc_v, sem):
        wid = wid_fn()
        pltpu.sync_copy(idx_hbm.at[:, wid, :], idx_v)

        def stream_chunk(c, carry):
            t0 = c * 10
            for u in range(10):
                pltpu.async_copy(
                    d_hbm.at[idx_v.at[t0 + u]],
                    g_v.at[t0 + u],
                    sem,
                )
            return carry

        lax.fori_loop(0, L // 10, stream_chunk, 0)
        for _ in range(L):
            pltpu.make_async_copy(
                d_hbm.at[pl.ds(0, TILE)], g_v.at[0], sem
            ).wait()

        def acc_j(j, accs):
            return tuple(
                accs[g] + g_v[j, pl.ds(g * 16, 16)]
                for g in range(NGRP)
            )

        zero = tuple(jnp.zeros((16,), jnp.float32) for _ in range(NGRP))
        accs = lax.fori_loop(0, L, acc_j, zero)
        for g in range(NGRP):
            acc_v[pl.ds(g * 16, 16)] = accs[g]
        pltpu.sync_copy(acc_v, out_hbm.at[pl.ds(wid * RPW, RPW)])

    return _gather_body


def _mesh_wid():
    return lax.axis_index("s") * NC + lax.axis_index("c")


_gm_scratch = [
    pltpu.VMEM((L, TILE), jnp.int32),
    pltpu.VMEM((L, TILE), jnp.float32),
    pltpu.VMEM((RPW,), jnp.float32),
    pltpu.SemaphoreType.DMA,
]

_gather = functools.partial(
    pl.kernel,
    mesh=_sc_mesh,
    out_type=jax.ShapeDtypeStruct((B,), jnp.float32),
    scratch_types=_gm_scratch,
    compiler_params=pltpu.CompilerParams(
        use_tc_tiling_on_sc=False, needs_layout_passes=False
    ),
)(_make_gather_body(_mesh_wid))


def _post_body(d_ref, lin_b_ref, s_emb_ref, rt_ref, apt_ref):
    d = d_ref[...] * (1.0 / L) + (lin_b_ref[0, 1] - lin_b_ref[0, 0])
    p1 = 1.0 / (1.0 + jnp.exp(-d))
    p0 = 1.0 - p1
    apt = jnp.concatenate([p0, p1], axis=0)
    apt_ref[...] = apt
    rt_ref[...] = lax.dot_general(
        s_emb_ref[...], apt,
        (((0,), (0,)), ((), ())),
        preferred_element_type=jnp.float32,
    )


_post = pl.pallas_call(
    _post_body,
    out_shape=(
        jax.ShapeDtypeStruct((EMB, B), jnp.float32),
        jax.ShapeDtypeStruct((NUM_SENS, B), jnp.float32),
    ),
)


@jax.jit
def kernel(inputs, asp_probs, W, lin_w, lin_b, s_emb):
    idxt = inputs.astype(jnp.int32).T
    d_table, idx3 = _proj(W.T, lin_w, idxt)
    d_sum = _gather(idx3, d_table).reshape(1, B)
    rt, apt = _post(d_sum, lin_b.reshape(1, NUM_SENS), s_emb)
    return (rt.T, apt.T)
